# baseline (speedup 1.0000x reference)
import jax
import jax.numpy as jnp
import numpy as np
from jax.experimental import pallas as pl
from jax.experimental.pallas import tpu as pltpu
from jax.experimental.shard_map import shard_map
from jax.sharding import Mesh, PartitionSpec as P

_MD = 2048
_HID = 1024
_TILE = 128


def _gelu(v):
    return 0.5 * v * (1.0 + jax.lax.erf(v * 0.7071067811865476))


def _ffn_body(x_ref, w1_ref, b1_ref, w2_ref, b2_ref, w3_ref, b3_ref,
              w4_ref, b4_ref, o_ref):
    h = jnp.dot(x_ref[...], w1_ref[...], preferred_element_type=jnp.float32)
    h = _gelu(h + b1_ref[...])
    y = jnp.dot(h, w2_ref[...], preferred_element_type=jnp.float32)
    y = y + b2_ref[...]
    g = jnp.dot(y, w3_ref[...], preferred_element_type=jnp.float32)
    g = _gelu(g + b3_ref[...])
    o = jnp.dot(g, w4_ref[...], preferred_element_type=jnp.float32)
    o_ref[...] = o + b4_ref[...]


def _run(xf, W1, b1r, W2, b2r, W3, b3r, W4, b4r):
    n = xf.shape[0]
    vmem = pl.BlockSpec(memory_space=pltpu.VMEM)
    return pl.pallas_call(
        _ffn_body,
        grid=(n // _TILE,),
        in_specs=[
            pl.BlockSpec((_TILE, _MD), lambda i: (i, 0)),
            vmem, vmem, vmem, vmem, vmem, vmem, vmem, vmem,
        ],
        out_specs=pl.BlockSpec((_TILE, _MD), lambda i: (i, 0)),
        out_shape=jax.ShapeDtypeStruct((n, _MD), jnp.float32),
        compiler_params=pltpu.CompilerParams(
            dimension_semantics=("arbitrary",),
        ),
    )(xf, W1, b1r, W2, b2r, W3, b3r, W4, b4r)


def kernel(x, W1, b1, W2, b2, W3, b3, W4, b4):
    B, S, D = x.shape
    N = B * S
    xf = x.reshape(N, D)
    b1r = b1.reshape(1, _HID)
    b2r = b2.reshape(1, _MD)
    b3r = b3.reshape(1, _MD)
    b4r = b4.reshape(1, _MD)
    args = (xf, W1, b1r, W2, b2r, W3, b3r, W4, b4r)

    devs = jax.devices()
    if len(devs) >= 2:
        mesh = Mesh(np.array(devs[:2]), ("dp",))
        rep = P(None, None)
        fn = shard_map(
            _run, mesh=mesh,
            in_specs=(P("dp", None),) + (rep,) * 8,
            out_specs=P("dp", None),
            check_rep=False,
        )
        out = fn(*args)
    else:
        out = _run(*args)
    return out.reshape(B, S, D)


# speedup vs baseline: 2.5567x; 2.5567x over previous
import jax
import jax.numpy as jnp
from jax.experimental import pallas as pl
from jax.experimental.pallas import tpu as pltpu

_MD = 2048
_HID = 1024
_TILE = 128


def _gelu(v):
    return 0.5 * v * (1.0 + jax.lax.erf(v * 0.7071067811865476))


def _ffn_body(x_ref, w1_ref, b1_ref, w2_ref, b2_ref, w3_ref, b3_ref,
              w4_ref, b4_ref, o_ref):
    h = jnp.dot(x_ref[...], w1_ref[...], preferred_element_type=jnp.float32)
    h = _gelu(h + b1_ref[...])
    y = jnp.dot(h, w2_ref[...], preferred_element_type=jnp.float32)
    y = y + b2_ref[...]
    g = jnp.dot(y, w3_ref[...], preferred_element_type=jnp.float32)
    g = _gelu(g + b3_ref[...])
    o = jnp.dot(g, w4_ref[...], preferred_element_type=jnp.float32)
    o_ref[...] = o + b4_ref[...]


def kernel(x, W1, b1, W2, b2, W3, b3, W4, b4):
    B, S, D = x.shape
    N = B * S
    xf = x.reshape(N, D)
    b1r = b1.reshape(1, _HID)
    b2r = b2.reshape(1, _MD)
    b3r = b3.reshape(1, _MD)
    b4r = b4.reshape(1, _MD)

    vmem = pl.BlockSpec(memory_space=pltpu.VMEM)
    out = pl.pallas_call(
        _ffn_body,
        grid=(N // _TILE,),
        in_specs=[
            pl.BlockSpec((_TILE, _MD), lambda i: (i, 0)),
            vmem, vmem, vmem, vmem, vmem, vmem, vmem, vmem,
        ],
        out_specs=pl.BlockSpec((_TILE, _MD), lambda i: (i, 0)),
        out_shape=jax.ShapeDtypeStruct((N, _MD), jnp.float32),
        compiler_params=pltpu.CompilerParams(
            dimension_semantics=("arbitrary",),
        ),
    )(xf, W1, b1r, W2, b2r, W3, b3r, W4, b4r)
    return out.reshape(B, S, D)
